# Initial kernel scaffold; baseline (speedup 1.0000x reference)
import jax
import jax.numpy as jnp
from jax.experimental import pallas as pl


def kernel(X, G, W1, b1, W2, b2, W3, b3):
    raise NotImplementedError("write your pallas kernel here")


# speedup vs baseline: 5.4151x; 5.4151x over previous
import functools

import jax
import jax.numpy as jnp
from jax import lax
from jax.experimental import pallas as pl
from jax.experimental.pallas import tpu as pltpu
from jax.experimental.pallas import tpu_sc as plsc

_NW = 32
_IDXB = 128


def _mm_body(x_ref, w_ref, o_ref):
    o_ref[...] = jnp.dot(x_ref[...], w_ref[...],
                         preferred_element_type=jnp.float32)


def _matmul(x, w, block_rows):
    m, kd = x.shape
    _, nd = w.shape
    return pl.pallas_call(
        _mm_body,
        grid=(m // block_rows,),
        in_specs=[pl.BlockSpec((block_rows, kd), lambda i: (i, 0)),
                  pl.BlockSpec((kd, nd), lambda i: (0, 0))],
        out_specs=pl.BlockSpec((block_rows, nd), lambda i: (i, 0)),
        out_shape=jax.ShapeDtypeStruct((m, nd), jnp.float32),
    )(x, w)


def _head_body(h_ref, w_ref, b_ref, o_ref):
    logits = jnp.dot(h_ref[...], w_ref[...],
                     preferred_element_type=jnp.float32) + b_ref[...]
    mx = jnp.max(logits, axis=-1, keepdims=True)
    e = jnp.exp(logits - mx)
    o_ref[...] = e / jnp.sum(e, axis=-1, keepdims=True)


def _head(h, w3, b3, block_rows):
    m, hd = h.shape
    _, nc = w3.shape
    return pl.pallas_call(
        _head_body,
        grid=(m // block_rows,),
        in_specs=[pl.BlockSpec((block_rows, hd), lambda i: (i, 0)),
                  pl.BlockSpec((hd, nc), lambda i: (0, 0)),
                  pl.BlockSpec((1, nc), lambda i: (0, 0))],
        out_specs=pl.BlockSpec((block_rows, nc), lambda i: (i, 0)),
        out_shape=jax.ShapeDtypeStruct((m, nc), jnp.float32),
    )(h, w3, b3.reshape(1, nc))


def _gs_body(table_ref, g_ref, bias_ref, out_ref,
             idx_v, buf_v, out_v, bias_v, sem, *, gpw, k, w):
    wid = lax.axis_index("s") * 2 + lax.axis_index("c")
    npg = _IDXB // k
    pltpu.sync_copy(g_ref.at[pl.ds(wid * gpw, gpw)], idx_v)
    pltpu.sync_copy(bias_ref, bias_v)
    bias = [bias_v[pl.ds(c * 16, 16)] for c in range(w // 16)]
    lane = lax.iota(jnp.int32, 16)
    def idx_body(j, carry):
        row = idx_v.at[j]
        for c in range(_IDXB // 16):
            off = (c * 16) % k
            row[pl.ds(c * 16, 16)] = row[pl.ds(c * 16, 16)] * k + lane + off
        return carry
    lax.fori_loop(0, gpw, idx_body, 0)
    def grp_body(j, carry):
        pltpu.async_copy(table_ref.at[idx_v.at[j]], buf_v, sem).wait()
        for t in range(npg):
            acc = list(bias)
            for kk in range(k):
                row = buf_v.at[t * k + kk]
                for c in range(w // 16):
                    acc[c] = acc[c] + row[pl.ds(c * 16, 16)]
            node = (j * npg + t) * w
            for c in range(w // 16):
                out_v[pl.ds(node + c * 16, 16)] = acc[c]
        return carry
    lax.fori_loop(0, gpw, grp_body, 0)
    pltpu.sync_copy(out_v, out_ref.at[pl.ds(wid * gpw * _IDXB // k * w,
                                            gpw * _IDXB // k * w)])


def _gather_sum(table2, g2, bias, k):
    w = table2.shape[1]
    gpw = g2.shape[0] // _NW
    npw = gpw * _IDXB // k
    mesh = plsc.VectorSubcoreMesh(core_axis_name="c", subcore_axis_name="s")
    kfn = pl.kernel(
        functools.partial(_gs_body, gpw=gpw, k=k, w=w),
        out_type=jax.ShapeDtypeStruct((_NW * npw * w,), jnp.float32),
        mesh=mesh,
        scratch_types=[
            pltpu.VMEM((gpw, _IDXB), jnp.int32),
            pltpu.VMEM((_IDXB, w), jnp.float32),
            pltpu.VMEM((npw * w,), jnp.float32),
            pltpu.VMEM((w,), jnp.float32),
            pltpu.SemaphoreType.DMA,
        ],
        compiler_params=pltpu.CompilerParams(use_tc_tiling_on_sc=False),
    )
    return kfn(table2, g2, bias)


def kernel(X, G, W1, b1, W2, b2, W3, b3):
    n, d = X.shape
    k = G.shape[1]
    h1d, h2d = W1.shape[1], W2.shape[1]
    ncls = W3.shape[1]

    npad = -(-n // _NW // 8) * _NW * 8
    gp = jnp.pad(G, ((0, npad - n), (0, 0)))
    g2 = gp.reshape(npad * k // _IDXB, _IDXB)

    w1r = W1.reshape(k, d, h1d).transpose(1, 0, 2).reshape(d, k * h1d)
    w2r = W2.reshape(k, h1d, h2d).transpose(1, 0, 2).reshape(h1d, k * h2d)

    p = _matmul(X, w1r, 1000)
    h1 = _gather_sum(p.reshape(n * k, h1d), g2, b1, k)
    h1 = h1.reshape(npad, h1d)
    q = _matmul(h1, w2r, npad // 10)
    h2 = _gather_sum(q.reshape(npad * k, h2d), g2, b2, k)
    h2 = h2.reshape(npad, h2d)
    out = _head(h2, W3, b3, npad // 10)
    return out[:n]
